# baseline (speedup 1.0000x reference)
import functools

import jax
import jax.numpy as jnp
from jax import lax
from jax.experimental import pallas as pl
from jax.experimental.pallas import tpu as pltpu
from jax.experimental.pallas import tpu_sc as plsc

VOCAB = 1000000
HIDDEN = 64
MAX_LEN = 512
BATCH = 4096
SEQ = 200

NC = 2
NS = 16
NW = NC * NS

N_ROWS = BATCH * SEQ
N_PACKED = N_ROWS // 2
PK_PER_W = N_PACKED // NW
_TB = 32
GROUP = _TB * SEQ
CP = 200
N_CHUNKS = PK_PER_W // CP
N_PAIRS = N_CHUNKS // 2


def _make_gather():
    mesh = plsc.VectorSubcoreMesh(core_axis_name="c", subcore_axis_name="s",
                                  num_cores=NC, num_subcores=NS)

    @functools.partial(
        pl.kernel,
        out_type=jax.ShapeDtypeStruct((N_PACKED, 2 * HIDDEN), jnp.float32),
        mesh=mesh,
        scratch_types=[
            pltpu.VMEM((CP,), jnp.int32),
            pltpu.VMEM((CP,), jnp.int32),
            pltpu.VMEM((CP,), jnp.int32),
            pltpu.VMEM((CP,), jnp.int32),
            pltpu.VMEM((CP, 2 * HIDDEN), jnp.float32),
            pltpu.VMEM((CP, 2 * HIDDEN), jnp.float32),
            pltpu.VMEM((CP, 2 * HIDDEN), jnp.float32),
            pltpu.VMEM((CP, 2 * HIDDEN), jnp.float32),
            pltpu.SemaphoreType.DMA,
            pltpu.SemaphoreType.DMA,
        ],
        compiler_params=pltpu.CompilerParams(use_tc_tiling_on_sc=False),
    )
    def gather_k(table_hbm, idx_hbm, out_hbm,
                 idxl0, idxr0, idxl1, idxr1,
                 rowsl0, rowsr0, rowsl1, rowsr1, sem0, sem1):
        wid = lax.axis_index("s") * NC + lax.axis_index("c")
        w_base = wid * PK_PER_W

        def fire(pbase, idxl, idxr, rowsl, rowsr, sem):
            lo = pbase + (pbase // GROUP) * GROUP
            pltpu.sync_copy(idx_hbm.at[pl.ds(lo, CP)], idxl)
            pltpu.sync_copy(idx_hbm.at[pl.ds(lo + GROUP, CP)], idxr)
            pltpu.async_copy(table_hbm.at[idxl], rowsl, sem)
            pltpu.async_copy(table_hbm.at[idxr], rowsr, sem)

        def drain(pbase, idxl, idxr, rowsl, rowsr, sem):
            pltpu.make_async_copy(table_hbm.at[idxl], rowsl, sem).wait()
            pltpu.make_async_copy(table_hbm.at[idxr], rowsr, sem).wait()
            pltpu.sync_copy(rowsl.at[:, pl.ds(0, HIDDEN)],
                            out_hbm.at[pl.ds(pbase, CP), pl.ds(0, HIDDEN)])
            pltpu.sync_copy(rowsr.at[:, pl.ds(0, HIDDEN)],
                            out_hbm.at[pl.ds(pbase, CP), pl.ds(HIDDEN, HIDDEN)])

        fire(w_base, idxl0, idxr0, rowsl0, rowsr0, sem0)

        def body(j, carry):
            pbase0 = w_base + 2 * j * CP
            pbase1 = pbase0 + CP
            fire(pbase1, idxl1, idxr1, rowsl1, rowsr1, sem1)
            drain(pbase0, idxl0, idxr0, rowsl0, rowsr0, sem0)

            @pl.when(j + 1 < N_PAIRS)
            def _():
                fire(pbase1 + CP, idxl0, idxr0, rowsl0, rowsr0, sem0)

            drain(pbase1, idxl1, idxr1, rowsl1, rowsr1, sem1)
            return carry

        lax.fori_loop(0, N_PAIRS, body, 0)

    return gather_k


_gather = _make_gather()

_BB = _TB
_HB = BATCH // 2
_NK = _HB // _BB
_R = _BB * SEQ
_H2 = 2 * HIDDEN


def _ln_block(emb_ref, pos_ref, gamma_ref, beta_ref, out_ref):
    i = lax.broadcasted_iota(jnp.int32, (_H2, _H2), 0)
    j = lax.broadcasted_iota(jnp.int32, (_H2, _H2), 1)
    a = ((i < HIDDEN) == (j < HIDDEN)).astype(jnp.float32)

    x = emb_ref[...] + pos_ref[...]
    s = jax.lax.dot(x, a)
    q = jax.lax.dot(x * x, a)
    m = s * (1.0 / HIDDEN)
    var = q * (1.0 / HIDDEN) - m * m
    inv = lax.rsqrt(var + 1e-5)
    o = (x - m) * inv * gamma_ref[...] + beta_ref[...]
    out_ref[...] = jnp.concatenate([o[:, :HIDDEN], o[:, HIDDEN:]], axis=0)


def _ln(emb2, pos2t, gamma2, beta2):
    return pl.pallas_call(
        _ln_block,
        grid=(_NK,),
        in_specs=[
            pl.BlockSpec((_R, _H2), lambda k: (k, 0)),
            pl.BlockSpec((_R, _H2), lambda k: (0, 0)),
            pl.BlockSpec((_H2,), lambda k: (0,)),
            pl.BlockSpec((_H2,), lambda k: (0,)),
        ],
        out_specs=pl.BlockSpec((2 * _R, HIDDEN), lambda k: (k, 0)),
        out_shape=jax.ShapeDtypeStruct((N_ROWS, HIDDEN), jnp.float32),
    )(emb2, pos2t, gamma2, beta2)


def kernel(input_ids, word_table, pos_table, gamma, beta):
    ids = input_ids.astype(jnp.int32)
    tbl128 = jnp.pad(word_table, ((0, 0), (0, HIDDEN)))
    emb2 = _gather(tbl128, ids.reshape(-1))
    pos = pos_table[:SEQ]
    pos2t = jnp.tile(jnp.concatenate([pos, pos], axis=1), (_BB, 1))
    gamma2 = jnp.concatenate([gamma, gamma])
    beta2 = jnp.concatenate([beta, beta])
    out = _ln(emb2, pos2t, gamma2, beta2)
    return out.reshape(BATCH, SEQ, HIDDEN)


# speedup vs baseline: 1.0344x; 1.0344x over previous
import functools

import jax
import jax.numpy as jnp
from jax import lax
from jax.experimental import pallas as pl
from jax.experimental.pallas import tpu as pltpu
from jax.experimental.pallas import tpu_sc as plsc

VOCAB = 1000000
HIDDEN = 64
MAX_LEN = 512
BATCH = 4096
SEQ = 200

NC = 2
NS = 16
NW = NC * NS

N_ROWS = BATCH * SEQ
N_PACKED = N_ROWS // 2
PK_PER_W = N_PACKED // NW
_TB = 32
GROUP = _TB * SEQ
CP = 400
N_CHUNKS = PK_PER_W // CP
N_PAIRS = N_CHUNKS // 2


def _make_gather():
    mesh = plsc.VectorSubcoreMesh(core_axis_name="c", subcore_axis_name="s",
                                  num_cores=NC, num_subcores=NS)

    @functools.partial(
        pl.kernel,
        out_type=jax.ShapeDtypeStruct((N_PACKED, 2 * HIDDEN), jnp.float32),
        mesh=mesh,
        scratch_types=[
            pltpu.VMEM((CP,), jnp.int32),
            pltpu.VMEM((CP,), jnp.int32),
            pltpu.VMEM((CP,), jnp.int32),
            pltpu.VMEM((CP,), jnp.int32),
            pltpu.VMEM((CP, HIDDEN), jnp.float32),
            pltpu.VMEM((CP, HIDDEN), jnp.float32),
            pltpu.VMEM((CP, HIDDEN), jnp.float32),
            pltpu.VMEM((CP, HIDDEN), jnp.float32),
            pltpu.SemaphoreType.DMA,
            pltpu.SemaphoreType.DMA,
        ],
        compiler_params=pltpu.CompilerParams(use_tc_tiling_on_sc=False),
    )
    def gather_k(table_hbm, idx_hbm, out_hbm,
                 idxl0, idxr0, idxl1, idxr1,
                 rowsl0, rowsr0, rowsl1, rowsr1, sem0, sem1):
        wid = lax.axis_index("s") * NC + lax.axis_index("c")
        w_base = wid * PK_PER_W

        def fire(pbase, idxl, idxr, rowsl, rowsr, sem):
            lo = pbase + (pbase // GROUP) * GROUP
            pltpu.sync_copy(idx_hbm.at[pl.ds(lo, CP)], idxl)
            pltpu.sync_copy(idx_hbm.at[pl.ds(lo + GROUP, CP)], idxr)
            pltpu.async_copy(table_hbm.at[idxl], rowsl, sem)
            pltpu.async_copy(table_hbm.at[idxr], rowsr, sem)

        def drain(pbase, idxl, idxr, rowsl, rowsr, sem):
            pltpu.make_async_copy(table_hbm.at[idxl], rowsl, sem).wait()
            pltpu.make_async_copy(table_hbm.at[idxr], rowsr, sem).wait()
            pltpu.sync_copy(rowsl,
                            out_hbm.at[pl.ds(pbase, CP), pl.ds(0, HIDDEN)])
            pltpu.sync_copy(rowsr,
                            out_hbm.at[pl.ds(pbase, CP), pl.ds(HIDDEN, HIDDEN)])

        fire(w_base, idxl0, idxr0, rowsl0, rowsr0, sem0)

        def body(j, carry):
            pbase0 = w_base + 2 * j * CP
            pbase1 = pbase0 + CP
            fire(pbase1, idxl1, idxr1, rowsl1, rowsr1, sem1)
            drain(pbase0, idxl0, idxr0, rowsl0, rowsr0, sem0)

            @pl.when(j + 1 < N_PAIRS)
            def _():
                fire(pbase1 + CP, idxl0, idxr0, rowsl0, rowsr0, sem0)

            drain(pbase1, idxl1, idxr1, rowsl1, rowsr1, sem1)
            return carry

        lax.fori_loop(0, N_PAIRS, body, 0)

    return gather_k


_gather = _make_gather()

_BB = _TB
_HB = BATCH // 2
_NK = _HB // _BB
_R = _BB * SEQ
_H2 = 2 * HIDDEN


def _ln_block(emb_ref, pos_ref, gamma_ref, beta_ref, out_ref):
    i = lax.broadcasted_iota(jnp.int32, (_H2, _H2), 0)
    j = lax.broadcasted_iota(jnp.int32, (_H2, _H2), 1)
    a = ((i < HIDDEN) == (j < HIDDEN)).astype(jnp.float32)

    x = emb_ref[...] + pos_ref[...]
    s = jax.lax.dot(x, a)
    q = jax.lax.dot(x * x, a)
    m = s * (1.0 / HIDDEN)
    var = q * (1.0 / HIDDEN) - m * m
    inv = lax.rsqrt(var + 1e-5)
    o = (x - m) * inv * gamma_ref[...] + beta_ref[...]
    out_ref[...] = jnp.concatenate([o[:, :HIDDEN], o[:, HIDDEN:]], axis=0)


def _ln(emb2, pos2t, gamma2, beta2):
    return pl.pallas_call(
        _ln_block,
        grid=(_NK,),
        in_specs=[
            pl.BlockSpec((_R, _H2), lambda k: (k, 0)),
            pl.BlockSpec((_R, _H2), lambda k: (0, 0)),
            pl.BlockSpec((_H2,), lambda k: (0,)),
            pl.BlockSpec((_H2,), lambda k: (0,)),
        ],
        out_specs=pl.BlockSpec((2 * _R, HIDDEN), lambda k: (k, 0)),
        out_shape=jax.ShapeDtypeStruct((N_ROWS, HIDDEN), jnp.float32),
    )(emb2, pos2t, gamma2, beta2)


def kernel(input_ids, word_table, pos_table, gamma, beta):
    ids = input_ids.astype(jnp.int32)
    emb2 = _gather(word_table, ids.reshape(-1))
    pos = pos_table[:SEQ]
    pos2t = jnp.tile(jnp.concatenate([pos, pos], axis=1), (_BB, 1))
    gamma2 = jnp.concatenate([gamma, gamma])
    beta2 = jnp.concatenate([beta, beta])
    out = _ln(emb2, pos2t, gamma2, beta2)
    return out.reshape(BATCH, SEQ, HIDDEN)


# speedup vs baseline: 1.0392x; 1.0046x over previous
import functools

import jax
import jax.numpy as jnp
from jax import lax
from jax.experimental import pallas as pl
from jax.experimental.pallas import tpu as pltpu
from jax.experimental.pallas import tpu_sc as plsc

VOCAB = 1000000
HIDDEN = 64
MAX_LEN = 512
BATCH = 4096
SEQ = 200

NC = 2
NS = 16
NW = NC * NS

N_ROWS = BATCH * SEQ
N_PACKED = N_ROWS // 2
PK_PER_W = N_PACKED // NW
_TB = 64
GROUP = _TB * SEQ
CP = 400
N_CHUNKS = PK_PER_W // CP
N_PAIRS = N_CHUNKS // 2


def _make_gather():
    mesh = plsc.VectorSubcoreMesh(core_axis_name="c", subcore_axis_name="s",
                                  num_cores=NC, num_subcores=NS)

    @functools.partial(
        pl.kernel,
        out_type=jax.ShapeDtypeStruct((N_PACKED, 2 * HIDDEN), jnp.float32),
        mesh=mesh,
        scratch_types=[
            pltpu.VMEM((CP,), jnp.int32),
            pltpu.VMEM((CP,), jnp.int32),
            pltpu.VMEM((CP,), jnp.int32),
            pltpu.VMEM((CP,), jnp.int32),
            pltpu.VMEM((CP, HIDDEN), jnp.float32),
            pltpu.VMEM((CP, HIDDEN), jnp.float32),
            pltpu.VMEM((CP, HIDDEN), jnp.float32),
            pltpu.VMEM((CP, HIDDEN), jnp.float32),
            pltpu.SemaphoreType.DMA,
            pltpu.SemaphoreType.DMA,
        ],
        compiler_params=pltpu.CompilerParams(use_tc_tiling_on_sc=False),
    )
    def gather_k(table_hbm, idx_hbm, out_hbm,
                 idxl0, idxr0, idxl1, idxr1,
                 rowsl0, rowsr0, rowsl1, rowsr1, sem0, sem1):
        wid = lax.axis_index("s") * NC + lax.axis_index("c")
        w_base = wid * PK_PER_W

        def fire(pbase, idxl, idxr, rowsl, rowsr, sem):
            lo = pbase + (pbase // GROUP) * GROUP
            pltpu.sync_copy(idx_hbm.at[pl.ds(lo, CP)], idxl)
            pltpu.sync_copy(idx_hbm.at[pl.ds(lo + GROUP, CP)], idxr)
            pltpu.async_copy(table_hbm.at[idxl], rowsl, sem)
            pltpu.async_copy(table_hbm.at[idxr], rowsr, sem)

        def drain(pbase, idxl, idxr, rowsl, rowsr, sem):
            pltpu.make_async_copy(table_hbm.at[idxl], rowsl, sem).wait()
            pltpu.make_async_copy(table_hbm.at[idxr], rowsr, sem).wait()
            pltpu.sync_copy(rowsl,
                            out_hbm.at[pl.ds(pbase, CP), pl.ds(0, HIDDEN)])
            pltpu.sync_copy(rowsr,
                            out_hbm.at[pl.ds(pbase, CP), pl.ds(HIDDEN, HIDDEN)])

        fire(w_base, idxl0, idxr0, rowsl0, rowsr0, sem0)

        def body(j, carry):
            pbase0 = w_base + 2 * j * CP
            pbase1 = pbase0 + CP
            fire(pbase1, idxl1, idxr1, rowsl1, rowsr1, sem1)
            drain(pbase0, idxl0, idxr0, rowsl0, rowsr0, sem0)

            @pl.when(j + 1 < N_PAIRS)
            def _():
                fire(pbase1 + CP, idxl0, idxr0, rowsl0, rowsr0, sem0)

            drain(pbase1, idxl1, idxr1, rowsl1, rowsr1, sem1)
            return carry

        lax.fori_loop(0, N_PAIRS, body, 0)

    return gather_k


_gather = _make_gather()

_BB = _TB
_HB = BATCH // 2
_NK = _HB // _BB
_R = _BB * SEQ
_H2 = 2 * HIDDEN


def _ln_block(emb_ref, pos_ref, gamma_ref, beta_ref, out_ref):
    i = lax.broadcasted_iota(jnp.int32, (_H2, _H2), 0)
    j = lax.broadcasted_iota(jnp.int32, (_H2, _H2), 1)
    a = ((i < HIDDEN) == (j < HIDDEN)).astype(jnp.float32)

    x = emb_ref[...] + pos_ref[...]
    s = jax.lax.dot(x, a)
    q = jax.lax.dot(x * x, a)
    m = s * (1.0 / HIDDEN)
    var = q * (1.0 / HIDDEN) - m * m
    inv = lax.rsqrt(var + 1e-5)
    o = (x - m) * inv * gamma_ref[...] + beta_ref[...]
    out_ref[...] = jnp.concatenate([o[:, :HIDDEN], o[:, HIDDEN:]], axis=0)


def _ln(emb2, pos2t, gamma2, beta2):
    return pl.pallas_call(
        _ln_block,
        grid=(_NK,),
        in_specs=[
            pl.BlockSpec((_R, _H2), lambda k: (k, 0)),
            pl.BlockSpec((_R, _H2), lambda k: (0, 0)),
            pl.BlockSpec((_H2,), lambda k: (0,)),
            pl.BlockSpec((_H2,), lambda k: (0,)),
        ],
        out_specs=pl.BlockSpec((2 * _R, HIDDEN), lambda k: (k, 0)),
        out_shape=jax.ShapeDtypeStruct((N_ROWS, HIDDEN), jnp.float32),
    )(emb2, pos2t, gamma2, beta2)


def kernel(input_ids, word_table, pos_table, gamma, beta):
    ids = input_ids.astype(jnp.int32)
    emb2 = _gather(word_table, ids.reshape(-1))
    pos = pos_table[:SEQ]
    pos2t = jnp.tile(jnp.concatenate([pos, pos], axis=1), (_BB, 1))
    gamma2 = jnp.concatenate([gamma, gamma])
    beta2 = jnp.concatenate([beta, beta])
    out = _ln(emb2, pos2t, gamma2, beta2)
    return out.reshape(BATCH, SEQ, HIDDEN)
